# baseline (speedup 1.0000x reference)
import functools

import jax
import jax.numpy as jnp
from jax import lax
from jax.experimental import pallas as pl
from jax.experimental.pallas import tpu as pltpu
from jax.experimental.pallas import tpu_sc as plsc

DIM = 64
B = 4096 * 200
NC, NS = 2, 16
NW = NC * NS
B_PER_W = B // NW
SUB = 128
CH = 512
NSUB = CH // SUB
NCH = B_PER_W // CH
ROWS_2D_PER_W = B_PER_W // SUB

_mesh = plsc.VectorSubcoreMesh(core_axis_name="c", subcore_axis_name="s")


@functools.partial(
    pl.kernel,
    mesh=_mesh,
    out_type=jax.ShapeDtypeStruct((B, DIM), jnp.float32),
    scratch_types=[
        pltpu.VMEM((NSUB, SUB), jnp.int32),
        pltpu.VMEM((CH, DIM), jnp.float32),
        pltpu.SemaphoreType.DMA,
    ],
    compiler_params=pltpu.CompilerParams(use_tc_tiling_on_sc=False),
)
def _emb(idx_hbm, table_hbm, out_hbm, idx_v, rows_v, sem):
    wid = lax.axis_index("s") * NC + lax.axis_index("c")
    base = wid * B_PER_W
    base2d = wid * ROWS_2D_PER_W

    def chunk(g, carry):
        pltpu.sync_copy(idx_hbm.at[pl.ds(base2d + g * NSUB, NSUB)], idx_v)
        copies = [
            pltpu.async_copy(
                table_hbm.at[idx_v.at[j]],
                rows_v.at[pl.ds(j * SUB, SUB)],
                sem,
            )
            for j in range(NSUB)
        ]
        for cp in copies:
            cp.wait()
        pltpu.sync_copy(rows_v, out_hbm.at[pl.ds(base + g * CH, CH)])
        return carry

    lax.fori_loop(0, NCH, chunk, 0)


def kernel(token_ids, weight):
    idx = token_ids.reshape(B // SUB, SUB).astype(jnp.int32)
    out = _emb(idx, weight)
    return out.reshape(token_ids.shape[0], token_ids.shape[1], DIM)


# speedup vs baseline: 1.0451x; 1.0451x over previous
import functools

import jax
import jax.numpy as jnp
from jax import lax
from jax.experimental import pallas as pl
from jax.experimental.pallas import tpu as pltpu
from jax.experimental.pallas import tpu_sc as plsc

DIM = 64
B = 4096 * 200
NC, NS = 2, 16
NW = NC * NS
B_PER_W = B // NW
SUB = 128
CH = 512
NSUB = CH // SUB
NCH = B_PER_W // CH
NBUF = 2

_mesh = plsc.VectorSubcoreMesh(core_axis_name="c", subcore_axis_name="s")


@functools.partial(
    pl.kernel,
    mesh=_mesh,
    out_type=jax.ShapeDtypeStruct((B, DIM), jnp.float32),
    scratch_types=[
        pltpu.VMEM((NBUF, CH), jnp.int32),
        pltpu.VMEM((NBUF, CH, DIM), jnp.float32),
        pltpu.SemaphoreType.DMA((NBUF,)),
        pltpu.SemaphoreType.DMA,
        pltpu.SemaphoreType.DMA((NBUF,)),
    ],
    compiler_params=pltpu.CompilerParams(use_tc_tiling_on_sc=False),
)
def _emb(idx_hbm, table_hbm, out_hbm, idx_v, rows_v, sem_idx, sem_g, sem_out):
    wid = lax.axis_index("s") * NC + lax.axis_index("c")
    base = wid * B_PER_W
    def idx_copy(g, b):
        return pltpu.make_async_copy(
            idx_hbm.at[pl.ds(base + g * CH, CH)], idx_v.at[b],
            sem_idx.at[b],
        )

    def out_copy(g, b):
        return pltpu.make_async_copy(
            rows_v.at[b], out_hbm.at[pl.ds(base + g * CH, CH)], sem_out.at[b],
        )

    def gather(b):
        copies = [
            pltpu.async_copy(
                table_hbm.at[idx_v.at[b, pl.ds(j * SUB, SUB)]],
                rows_v.at[b, pl.ds(j * SUB, SUB)],
                sem_g,
            )
            for j in range(NSUB)
        ]
        for cp in copies:
            cp.wait()

    for b in range(NBUF):
        idx_copy(b, b).start()

    for b in range(NBUF):
        idx_copy(b, b).wait()
        gather(b)
        out_copy(b, b).start()
        idx_copy(b + NBUF, b).start()

    def step(i, carry):
        g0 = NBUF + i * NBUF
        for b in range(NBUF):
            g = g0 + b
            idx_copy(g, b).wait()
            out_copy(g - NBUF, b).wait()
            gather(b)
            out_copy(g, b).start()
            idx_copy(g + NBUF, b).start()
        return carry

    lax.fori_loop(0, (NCH - 2 * NBUF) // NBUF, step, 0)

    for b in range(NBUF):
        g = NCH - NBUF + b
        idx_copy(g, b).wait()
        out_copy(g - NBUF, b).wait()
        gather(b)
        out_copy(g, b).start()

    for b in range(NBUF):
        out_copy(NCH - NBUF + b, b).wait()


def kernel(token_ids, weight):
    idx = token_ids.reshape(B).astype(jnp.int32)
    out = _emb(idx, weight)
    return out.reshape(token_ids.shape[0], token_ids.shape[1], DIM)


# speedup vs baseline: 1.0463x; 1.0012x over previous
import functools

import jax
import jax.numpy as jnp
from jax import lax
from jax.experimental import pallas as pl
from jax.experimental.pallas import tpu as pltpu
from jax.experimental.pallas import tpu_sc as plsc

DIM = 64
B = 4096 * 200
NC, NS = 2, 16
NW = NC * NS
B_PER_W = B // NW
SUB = 128
CH = 256
NSUB = CH // SUB
NCH = B_PER_W // CH
NBUF = 4

_mesh = plsc.VectorSubcoreMesh(core_axis_name="c", subcore_axis_name="s")


@functools.partial(
    pl.kernel,
    mesh=_mesh,
    out_type=jax.ShapeDtypeStruct((B, DIM), jnp.float32),
    scratch_types=[
        pltpu.VMEM((NBUF, CH), jnp.int32),
        pltpu.VMEM((NBUF, CH, DIM), jnp.float32),
        pltpu.SemaphoreType.DMA((NBUF,)),
        pltpu.SemaphoreType.DMA,
        pltpu.SemaphoreType.DMA((NBUF,)),
    ],
    compiler_params=pltpu.CompilerParams(use_tc_tiling_on_sc=False),
)
def _emb(idx_hbm, table_hbm, out_hbm, idx_v, rows_v, sem_idx, sem_g, sem_out):
    wid = lax.axis_index("s") * NC + lax.axis_index("c")
    base = wid * B_PER_W

    def idx_copy(g, b):
        return pltpu.make_async_copy(
            idx_hbm.at[pl.ds(base + g * CH, CH)], idx_v.at[b], sem_idx.at[b],
        )

    def out_copy(g, b):
        return pltpu.make_async_copy(
            rows_v.at[b], out_hbm.at[pl.ds(base + g * CH, CH)], sem_out.at[b],
        )

    def fire_gathers(b):
        for j in range(NSUB):
            pltpu.async_copy(
                table_hbm.at[idx_v.at[b, pl.ds(j * SUB, SUB)]],
                rows_v.at[b, pl.ds(j * SUB, SUB)],
                sem_g,
            )

    def drain_gathers(b):
        for j in range(NSUB):
            pltpu.make_async_copy(
                table_hbm.at[idx_v.at[b, pl.ds(j * SUB, SUB)]],
                rows_v.at[b, pl.ds(j * SUB, SUB)],
                sem_g,
            ).wait()


    for b in range(NBUF):
        idx_copy(b, b).start()
    idx_copy(0, 0).wait()
    fire_gathers(0)

    def slot_head(g):
        idx_copy(g + 1, (g + 1) % NBUF).wait()
        fire_gathers((g + 1) % NBUF)
        drain_gathers(g % NBUF)
        idx_copy(g + NBUF, g % NBUF).start()
        out_copy(g, g % NBUF).start()

    def slot_steady(g):
        idx_copy(g + 1, (g + 1) % NBUF).wait()
        out_copy(g + 1 - NBUF, (g + 1) % NBUF).wait()
        fire_gathers((g + 1) % NBUF)
        drain_gathers(g % NBUF)
        idx_copy(g + NBUF, g % NBUF).start()
        out_copy(g, g % NBUF).start()

    def slot_tail(g, last):
        if not last:
            idx_copy(g + 1, (g + 1) % NBUF).wait()
            out_copy(g + 1 - NBUF, (g + 1) % NBUF).wait()
            fire_gathers((g + 1) % NBUF)
        drain_gathers(g % NBUF)
        out_copy(g, g % NBUF).start()

    for g in range(NBUF):
        if g < NBUF - 1:
            slot_head(g)
        else:
            slot_steady(g)

    def step(i, carry):
        g0 = NBUF + i * NBUF
        for b in range(NBUF):
            slot_steady(g0 + b)
        return carry

    lax.fori_loop(0, (NCH - 2 * NBUF) // NBUF, step, 0)

    for g in range(NCH - NBUF, NCH):
        slot_tail(g, last=(g == NCH - 1))

    for g in range(NCH - NBUF, NCH):
        out_copy(g, g % NBUF).wait()


def kernel(token_ids, weight):
    idx = token_ids.reshape(B).astype(jnp.int32)
    out = _emb(idx, weight)
    return out.reshape(token_ids.shape[0], token_ids.shape[1], DIM)


# speedup vs baseline: 1.0464x; 1.0001x over previous
import functools

import jax
import jax.numpy as jnp
from jax import lax
from jax.experimental import pallas as pl
from jax.experimental.pallas import tpu as pltpu
from jax.experimental.pallas import tpu_sc as plsc

DIM = 64
NSEQ = 4096
SEQ = 200
B = NSEQ * SEQ
NC, NS = 2, 16
NW = NC * NS
SEQ_PER_W = NSEQ // NW
CH = SEQ
NCH = SEQ_PER_W
SUBS = (128, 72)
NBUF = 4

_mesh = plsc.VectorSubcoreMesh(core_axis_name="c", subcore_axis_name="s")


@functools.partial(
    pl.kernel,
    mesh=_mesh,
    out_type=jax.ShapeDtypeStruct((NSEQ, SEQ, DIM), jnp.float32),
    scratch_types=[
        pltpu.VMEM((NBUF, CH), jnp.int32),
        pltpu.VMEM((NBUF, CH, DIM), jnp.float32),
        pltpu.SemaphoreType.DMA((NBUF,)),
        pltpu.SemaphoreType.DMA,
        pltpu.SemaphoreType.DMA((NBUF,)),
    ],
    compiler_params=pltpu.CompilerParams(use_tc_tiling_on_sc=False),
)
def _emb(idx_hbm, table_hbm, out_hbm, idx_v, rows_v, sem_idx, sem_g, sem_out):
    wid = lax.axis_index("s") * NC + lax.axis_index("c")
    seq_base = wid * SEQ_PER_W
    base = seq_base * SEQ

    def idx_copy(g, b):
        return pltpu.make_async_copy(
            idx_hbm.at[pl.ds(base + g * CH, CH)], idx_v.at[b], sem_idx.at[b],
        )

    def out_copy(g, b):
        return pltpu.make_async_copy(
            rows_v.at[b], out_hbm.at[seq_base + g], sem_out.at[b],
        )

    def gather_copies(b):
        off = 0
        copies = []
        for n in SUBS:
            copies.append(pltpu.make_async_copy(
                table_hbm.at[idx_v.at[b, pl.ds(off, n)]],
                rows_v.at[b, pl.ds(off, n)],
                sem_g,
            ))
            off += n
        return copies

    def fire_gathers(b):
        for cp in gather_copies(b):
            cp.start()

    def drain_gathers(b):
        for cp in gather_copies(b):
            cp.wait()


    for b in range(NBUF):
        idx_copy(b, b).start()
    idx_copy(0, 0).wait()
    fire_gathers(0)

    def slot_head(g):
        idx_copy(g + 1, (g + 1) % NBUF).wait()
        fire_gathers((g + 1) % NBUF)
        drain_gathers(g % NBUF)
        idx_copy(g + NBUF, g % NBUF).start()
        out_copy(g, g % NBUF).start()

    def slot_steady(g):
        idx_copy(g + 1, (g + 1) % NBUF).wait()
        out_copy(g + 1 - NBUF, (g + 1) % NBUF).wait()
        fire_gathers((g + 1) % NBUF)
        drain_gathers(g % NBUF)
        idx_copy(g + NBUF, g % NBUF).start()
        out_copy(g, g % NBUF).start()

    def slot_tail(g, last):
        if not last:
            idx_copy(g + 1, (g + 1) % NBUF).wait()
            out_copy(g + 1 - NBUF, (g + 1) % NBUF).wait()
            fire_gathers((g + 1) % NBUF)
        drain_gathers(g % NBUF)
        out_copy(g, g % NBUF).start()

    for g in range(NBUF):
        if g < NBUF - 1:
            slot_head(g)
        else:
            slot_steady(g)

    def step(i, carry):
        g0 = NBUF + i * NBUF
        for b in range(NBUF):
            slot_steady(g0 + b)
        return carry

    lax.fori_loop(0, (NCH - 2 * NBUF) // NBUF, step, 0)

    for g in range(NCH - NBUF, NCH):
        slot_tail(g, last=(g == NCH - 1))

    for g in range(NCH - NBUF, NCH):
        out_copy(g, g % NBUF).wait()


def kernel(token_ids, weight):
    idx = token_ids.reshape(B).astype(jnp.int32)
    return _emb(idx, weight)
